# Initial kernel scaffold; baseline (speedup 1.0000x reference)
import jax
import jax.numpy as jnp
from jax.experimental import pallas as pl


def kernel(pred_heatmap, pred_offset_2d, pred_size_2d, pred_vis_depth, pred_att_depth, pred_vis_depth_uncer, pred_att_depth_uncer, pred_ins_depth, pred_ins_depth_uncer, pred_offset_3d, pred_size_3d, pred_heading, tgt_heatmap, tgt_offset_2d, tgt_size_2d, tgt_depth, tgt_vis_depth, tgt_att_depth, tgt_offset_3d, tgt_size_3d, heading_res, indices, mask_2d, train_tag, heading_bin, depth_mask):
    raise NotImplementedError("write your pallas kernel here")


# speedup vs baseline: 1.0919x; 1.0919x over previous
import functools

import jax
import jax.numpy as jnp
from jax import lax
from jax.experimental import pallas as pl
from jax.experimental.pallas import tpu as pltpu
from jax.experimental.pallas import tpu_sc as plsc

B, K, H, W, NC = 16, 50, 96, 320, 3
N = B * K
HW = H * W
NIDX = 2 * N
NPAD = 2048
CHUNK = NPAD // 32

_BETA = 1.0 / 9.0


def _sl1(d):
    n = jnp.abs(d)
    return jnp.where(n < _BETA, 0.5 * n * n / _BETA, n - 0.5 * _BETA)


def _sc_gather_body(off_hbm, size_hbm, idx_hbm, out_off, out_size,
                    idx_v, val_a, val_b, sem_a, sem_b):
    c = lax.axis_index("c")
    s = lax.axis_index("s")
    info = plsc.get_sparse_core_info()
    wid = s * info.num_cores + c
    base = wid * CHUNK
    pltpu.sync_copy(idx_hbm.at[pl.ds(base, CHUNK)], idx_v)
    cp_a = pltpu.async_copy(off_hbm.at[idx_v], val_a, sem_a)
    cp_b = pltpu.async_copy(size_hbm.at[idx_v], val_b, sem_b)
    cp_a.wait()
    pltpu.sync_copy(val_a, out_off.at[pl.ds(base, CHUNK)])
    cp_b.wait()
    pltpu.sync_copy(val_b, out_size.at[pl.ds(base, CHUNK)])


def _sc_gather(off_flat, size_flat, idx):
    mesh = plsc.VectorSubcoreMesh(core_axis_name="c", subcore_axis_name="s")
    f = pl.kernel(
        _sc_gather_body,
        mesh=mesh,
        out_type=(
            jax.ShapeDtypeStruct((NPAD,), jnp.float32),
            jax.ShapeDtypeStruct((NPAD,), jnp.float32),
        ),
        scratch_types=[
            pltpu.VMEM((CHUNK,), jnp.int32),
            pltpu.VMEM((CHUNK,), jnp.float32),
            pltpu.VMEM((CHUNK,), jnp.float32),
            pltpu.SemaphoreType.DMA,
            pltpu.SemaphoreType.DMA,
        ],
    )
    return f(off_flat, size_flat, idx)


_HM_ROWS = B * NC
_HM_BLK = 8


def _tc_body(ph_ref, th_ref, s2in_ref, s2t_ref, o2in_ref, o2t_ref,
             fm_ref, tt_ref, dm_ref,
             vd_ref, vdt_ref, ad_ref, adt_ref, vu_ref, au_ref,
             ins_ref, insu_ref, idt_ref,
             o3in_ref, o3t_ref, s3in_ref, s3t_ref,
             hd_ref, tcls_ref, treg_ref,
             out_ref, acc_ref):
    i = pl.program_id(0)

    x = ph_ref[...]
    g = th_ref[...]
    p = jnp.clip(1.0 / (1.0 + jnp.exp(-x)), 1e-4, 1.0 - 1e-4)
    pos = (g == 1.0).astype(jnp.float32)
    neg = (g < 1.0).astype(jnp.float32)
    omg = 1.0 - g
    nw2 = omg * omg
    nw = nw2 * nw2
    omp = 1.0 - p
    pos_l = jnp.sum(jnp.log(p) * omp * omp * pos)
    neg_l = jnp.sum(jnp.log(omp) * p * p * nw * neg)
    npos = jnp.sum(pos)

    @pl.when(i == 0)
    def _init():
        acc_ref[0] = 0.0
        acc_ref[1] = 0.0
        acc_ref[2] = 0.0

    acc_ref[0] += pos_l
    acc_ref[1] += neg_l
    acc_ref[2] += npos

    @pl.when(i == pl.num_programs(0) - 1)
    def _final():
        pos_t = acc_ref[0]
        neg_t = acc_ref[1]
        npos_t = acc_ref[2]
        seg = jnp.where(npos_t == 0.0, -neg_t,
                        -(pos_t + neg_t) / jnp.maximum(npos_t, 1.0))

        cm = fm_ref[...]
        cbf = cm * tt_ref[...]
        cnt_m = jnp.sum(cm)
        cnt_b = jnp.sum(cbf)
        dmf = dm_ref[...] * cbf
        cnt_dm = jnp.sum(dmf)

        s2d = jnp.sum(jnp.abs(s2in_ref[...] - s2t_ref[...]) * cm) / (cnt_m * 2.0)
        o2d = jnp.sum(jnp.abs(o2in_ref[...] - o2t_ref[...]) * cm) / (cnt_m * 2.0)

        vu = vu_ref[...]
        au = au_ref[...]
        vis = jnp.sum((1.4142 * jnp.exp(-vu) * jnp.abs(vd_ref[...] - vdt_ref[...]) + vu)
                      * dmf) / cnt_dm
        att = jnp.sum((1.4142 * jnp.exp(-au) * jnp.abs(ad_ref[...] - adt_ref[...]) + au)
                      * dmf) / cnt_dm

        ins = ins_ref[...]
        insu = insu_ref[...]
        idt = idt_ref[...]
        ins_l = jnp.sum((1.4142 * jnp.exp(-insu) * jnp.abs(ins - idt) + insu)
                        * cbf) / (cnt_b * 49.0)
        mp = jnp.exp(-jnp.exp(0.5 * insu))
        md = (jnp.sum(ins * mp, axis=1, keepdims=True)
              / (jnp.sum(mp, axis=1, keepdims=True) + 1e-8))
        dw = jnp.exp(-jnp.abs(jnp.abs(md - idt) - 0.35))
        idt_w = jnp.where(idt != idt, md, idt)
        ins1 = jnp.sum(_sl1(md - idt_w) * dw * cbf) / cnt_b
        depth = vis + att + ins_l + ins1

        o3d = jnp.sum(jnp.abs(o3in_ref[...] - o3t_ref[...]) * cbf) / (cnt_b * 2.0)
        s3in = s3in_ref[...]
        s3t = s3t_ref[...]
        s3d = jnp.sum(jnp.abs(s3in - s3t) * cbf) / (cnt_b * 3.0)
        s3h_in = s3in[:, 2:3]
        s3h_t = s3t[:, 2:3]
        s3h_tw = jnp.where(s3h_t != s3h_t, s3h_in, s3h_t)
        s3d = s3d + jnp.sum(_sl1(s3h_in - s3h_tw) * dw * cbf) / cnt_b

        hd = hd_ref[...]
        logits = hd[:, 0:12]
        m = jnp.max(logits, axis=1, keepdims=True)
        z = logits - m
        lse = jnp.log(jnp.sum(jnp.exp(z), axis=1, keepdims=True))
        logp = z - lse
        oh = (lax.broadcasted_iota(jnp.int32, (N, 12), 1)
              == tcls_ref[...]).astype(jnp.float32)
        cls_l = -jnp.sum(jnp.sum(logp * oh, axis=1, keepdims=True) * cbf) / cnt_b
        reg_in = jnp.sum(hd[:, 12:24] * oh, axis=1, keepdims=True)
        reg_l = jnp.sum(jnp.abs(reg_in - treg_ref[...]) * cbf) / cnt_b
        heading = cls_l + reg_l

        out_ref[0, 0] = seg + o2d + s2d + depth + o3d + s3d + heading


def _full(shape):
    return pl.BlockSpec(shape, lambda i: (0, 0))


@functools.partial(jax.jit, static_argnums=())
def kernel(pred_heatmap, pred_offset_2d, pred_size_2d, pred_vis_depth,
           pred_att_depth, pred_vis_depth_uncer, pred_att_depth_uncer,
           pred_ins_depth, pred_ins_depth_uncer, pred_offset_3d, pred_size_3d,
           pred_heading, tgt_heatmap, tgt_offset_2d, tgt_size_2d, tgt_depth,
           tgt_vis_depth, tgt_att_depth, tgt_offset_3d, tgt_size_3d,
           heading_res, indices, mask_2d, train_tag, heading_bin, depth_mask):
    ind = indices.astype(jnp.int32)
    base = ind + (jnp.arange(B, dtype=jnp.int32) * (2 * HW))[:, None]
    idx = jnp.stack([base, base + HW], axis=-1).reshape(-1)
    idx = jnp.concatenate([idx, jnp.zeros((NPAD - NIDX,), jnp.int32)])
    off_g, size_g = _sc_gather(pred_offset_2d.reshape(-1),
                               pred_size_2d.reshape(-1), idx)
    o2d_in = off_g[:NIDX].reshape(N, 2)
    s2d_in = size_g[:NIDX].reshape(N, 2)

    ph = pred_heatmap.reshape(_HM_ROWS, HW)
    th = tgt_heatmap.reshape(_HM_ROWS, HW)
    grid = _HM_ROWS // _HM_BLK

    hm_spec = pl.BlockSpec((_HM_BLK, HW), lambda i: (i, 0))
    small_inputs = [
        s2d_in, tgt_size_2d.reshape(N, 2),
        o2d_in, tgt_offset_2d.reshape(N, 2),
        mask_2d.reshape(N, 1).astype(jnp.float32),
        train_tag.reshape(N, 1).astype(jnp.float32),
        depth_mask.reshape(N, 49).astype(jnp.float32),
        pred_vis_depth.reshape(N, 49), tgt_vis_depth.reshape(N, 49),
        pred_att_depth.reshape(N, 49), tgt_att_depth.reshape(N, 49),
        pred_vis_depth_uncer.reshape(N, 49), pred_att_depth_uncer.reshape(N, 49),
        pred_ins_depth.reshape(N, 49), pred_ins_depth_uncer.reshape(N, 49),
        tgt_depth.reshape(N, 1),
        pred_offset_3d, tgt_offset_3d.reshape(N, 2),
        pred_size_3d, tgt_size_3d.reshape(N, 3),
        pred_heading,
        heading_bin.reshape(N, 1).astype(jnp.int32),
        heading_res.reshape(N, 1),
    ]
    in_specs = [hm_spec, hm_spec] + [_full(a.shape) for a in small_inputs]

    out = pl.pallas_call(
        _tc_body,
        grid=(grid,),
        in_specs=in_specs,
        out_specs=pl.BlockSpec(memory_space=pltpu.SMEM),
        out_shape=jax.ShapeDtypeStruct((1, 1), jnp.float32),
        scratch_shapes=[pltpu.SMEM((3,), jnp.float32)],
    )(ph, th, *small_inputs)
    return jnp.reshape(out, ())


# speedup vs baseline: 1.6647x; 1.5246x over previous
import functools

import jax
import jax.numpy as jnp
from jax import lax
from jax.experimental import pallas as pl
from jax.experimental.pallas import tpu as pltpu
from jax.experimental.pallas import tpu_sc as plsc

B, K, H, W, NC = 16, 50, 96, 320, 3
N = B * K
HW = H * W
NIDX = 2 * N
NPAD = 2048
CHUNK = NPAD // 32

_BETA = 1.0 / 9.0
_CLIP = 9.210240366975849


def _sl1(d):
    n = jnp.abs(d)
    return jnp.where(n < _BETA, 0.5 * n * n / _BETA, n - 0.5 * _BETA)


def _sc_gather_body(off_hbm, size_hbm, idx_hbm, out, idx_v, val_a, val_b,
                    sem_a, sem_b):
    c = lax.axis_index("c")
    s = lax.axis_index("s")
    info = plsc.get_sparse_core_info()
    wid = s * info.num_cores + c
    base = wid * CHUNK
    pltpu.sync_copy(idx_hbm.at[pl.ds(base, CHUNK)], idx_v)
    cp_a = pltpu.async_copy(off_hbm.at[idx_v], val_a, sem_a)
    cp_b = pltpu.async_copy(size_hbm.at[idx_v], val_b, sem_b)
    cp_a.wait()
    pltpu.sync_copy(val_a, out.at[pl.ds(base, CHUNK)])
    cp_b.wait()
    pltpu.sync_copy(val_b, out.at[pl.ds(NPAD + base, CHUNK)])


def _sc_gather(off_flat, size_flat, idx):
    mesh = plsc.VectorSubcoreMesh(core_axis_name="c", subcore_axis_name="s")
    f = pl.kernel(
        _sc_gather_body,
        mesh=mesh,
        out_type=jax.ShapeDtypeStruct((2 * NPAD,), jnp.float32),
        scratch_types=[
            pltpu.VMEM((CHUNK,), jnp.int32),
            pltpu.VMEM((CHUNK,), jnp.float32),
            pltpu.VMEM((CHUNK,), jnp.float32),
            pltpu.SemaphoreType.DMA,
            pltpu.SemaphoreType.DMA,
        ],
    )
    return f(off_flat, size_flat, idx)


_HB = 8
_GRID = H // _HB

_C_CM, _C_TT, _C_IDT, _C_TREG, _C_TCLS = 0, 1, 2, 3, 4
_C_O3IN, _C_O3T, _C_S3IN, _C_S3T, _C_HD = 5, 7, 9, 12, 15


def _tc_body(ph_ref, th_ref, g2d_ref, t2d_ref, m2d_ref, a_ref, b_ref,
             out_ref, acc_ref):
    i = pl.program_id(0)

    x = jnp.clip(ph_ref[...], -_CLIP, _CLIP)
    g = th_ref[...]
    e = jnp.exp(-x)
    t = 1.0 + e
    p = 1.0 / t
    s = jnp.log(t)
    omp = e * p
    pos = (g == 1.0).astype(jnp.float32)
    neg = (g < 1.0).astype(jnp.float32)
    omg = 1.0 - g
    nw2 = omg * omg
    nw = nw2 * nw2
    pos_l = jnp.sum(s * omp * omp * pos)
    neg_l = jnp.sum((x + s) * p * p * nw * neg)
    npos = jnp.sum(pos)

    @pl.when(i == 0)
    def _init():
        acc_ref[0] = 0.0
        acc_ref[1] = 0.0
        acc_ref[2] = 0.0

    acc_ref[0] += pos_l
    acc_ref[1] += neg_l
    acc_ref[2] += npos

    @pl.when(i == pl.num_programs(0) - 1)
    def _final():
        seg = jnp.where(acc_ref[2] == 0.0, -(-acc_ref[1]),
                        -(-acc_ref[0] + -acc_ref[1])
                        / jnp.maximum(acc_ref[2], 1.0))

        bm = b_ref[...]
        cm = bm[:, _C_CM:_C_CM + 1]
        cbf = cm * bm[:, _C_TT:_C_TT + 1]
        idt = bm[:, _C_IDT:_C_IDT + 1]
        cnt_m = jnp.sum(cm)
        cnt_b = jnp.sum(cbf)
        dmf = a_ref[8] * cbf
        cnt_dm = jnp.sum(dmf)

        l2d = jnp.sum(jnp.abs(g2d_ref[...] - t2d_ref[...]) * m2d_ref[...]) \
            / (cnt_m * 2.0)

        vu = a_ref[4]
        au = a_ref[5]
        vis = jnp.sum((1.4142 * jnp.exp(-vu) * jnp.abs(a_ref[0] - a_ref[1])
                       + vu) * dmf) / cnt_dm
        att = jnp.sum((1.4142 * jnp.exp(-au) * jnp.abs(a_ref[2] - a_ref[3])
                       + au) * dmf) / cnt_dm

        ins = a_ref[6]
        insu = a_ref[7]
        ins_l = jnp.sum((1.4142 * jnp.exp(-insu) * jnp.abs(ins - idt) + insu)
                        * cbf) / (cnt_b * 49.0)
        mp = jnp.exp(-jnp.exp(0.5 * insu))
        md = (jnp.sum(ins * mp, axis=1, keepdims=True)
              / (jnp.sum(mp, axis=1, keepdims=True) + 1e-8))
        dw = jnp.exp(-jnp.abs(jnp.abs(md - idt) - 0.35))
        idt_w = jnp.where(idt != idt, md, idt)
        ins1 = jnp.sum(_sl1(md - idt_w) * dw * cbf) / cnt_b
        depth = vis + att + ins_l + ins1

        o3d = jnp.sum(jnp.abs(bm[:, _C_O3IN:_C_O3IN + 2]
                              - bm[:, _C_O3T:_C_O3T + 2]) * cbf) / (cnt_b * 2.0)
        s3in = bm[:, _C_S3IN:_C_S3IN + 3]
        s3t = bm[:, _C_S3T:_C_S3T + 3]
        s3d = jnp.sum(jnp.abs(s3in - s3t) * cbf) / (cnt_b * 3.0)
        s3h_in = s3in[:, 2:3]
        s3h_t = s3t[:, 2:3]
        s3h_tw = jnp.where(s3h_t != s3h_t, s3h_in, s3h_t)
        s3d = s3d + jnp.sum(_sl1(s3h_in - s3h_tw) * dw * cbf) / cnt_b

        hd = bm[:, _C_HD:_C_HD + 24]
        logits = hd[:, 0:12]
        mx = jnp.max(logits, axis=1, keepdims=True)
        z = logits - mx
        lse = jnp.log(jnp.sum(jnp.exp(z), axis=1, keepdims=True))
        logp = z - lse
        tclsf = bm[:, _C_TCLS:_C_TCLS + 1]
        oh = (lax.broadcasted_iota(jnp.int32, (N, 12), 1).astype(jnp.float32)
              == tclsf).astype(jnp.float32)
        cls_l = -jnp.sum(jnp.sum(logp * oh, axis=1, keepdims=True)
                         * cbf) / cnt_b
        reg_in = jnp.sum(hd[:, 12:24] * oh, axis=1, keepdims=True)
        reg_l = jnp.sum(jnp.abs(reg_in - bm[:, _C_TREG:_C_TREG + 1])
                        * cbf) / cnt_b

        out_ref[0, 0] = seg + l2d + depth + o3d + s3d + cls_l + reg_l


def _full2(shape):
    return pl.BlockSpec(shape, lambda i: (0, 0))


@functools.partial(jax.jit, static_argnums=())
def kernel(pred_heatmap, pred_offset_2d, pred_size_2d, pred_vis_depth,
           pred_att_depth, pred_vis_depth_uncer, pred_att_depth_uncer,
           pred_ins_depth, pred_ins_depth_uncer, pred_offset_3d, pred_size_3d,
           pred_heading, tgt_heatmap, tgt_offset_2d, tgt_size_2d, tgt_depth,
           tgt_vis_depth, tgt_att_depth, tgt_offset_3d, tgt_size_3d,
           heading_res, indices, mask_2d, train_tag, heading_bin, depth_mask):
    f32 = jnp.float32
    ind = indices.astype(jnp.int32)
    base = ind + (jnp.arange(B, dtype=jnp.int32) * (2 * HW))[:, None]
    idx = jnp.stack([base, base + HW], axis=-1).reshape(-1)
    idx = jnp.concatenate([idx, jnp.zeros((NPAD - NIDX,), jnp.int32)])
    g2d_flat = _sc_gather(pred_offset_2d.reshape(-1),
                          pred_size_2d.reshape(-1), idx)
    g2d = g2d_flat.reshape(32, 128)

    zpad = jnp.zeros((NPAD - NIDX,), f32)
    t2d = jnp.concatenate([tgt_offset_2d.reshape(-1), zpad,
                           tgt_size_2d.reshape(-1), zpad]).reshape(32, 128)
    mrep = jnp.repeat(mask_2d.reshape(-1).astype(f32), 2)
    m2d = jnp.concatenate([mrep, zpad, mrep, zpad]).reshape(32, 128)

    a_pack = jnp.stack([
        pred_vis_depth.reshape(N, 49), tgt_vis_depth.reshape(N, 49),
        pred_att_depth.reshape(N, 49), tgt_att_depth.reshape(N, 49),
        pred_vis_depth_uncer.reshape(N, 49), pred_att_depth_uncer.reshape(N, 49),
        pred_ins_depth.reshape(N, 49), pred_ins_depth_uncer.reshape(N, 49),
        depth_mask.reshape(N, 49).astype(f32),
    ])
    b_pack = jnp.concatenate([
        mask_2d.reshape(N, 1).astype(f32),
        train_tag.reshape(N, 1).astype(f32),
        tgt_depth.reshape(N, 1),
        heading_res.reshape(N, 1),
        heading_bin.reshape(N, 1).astype(f32),
        pred_offset_3d, tgt_offset_3d.reshape(N, 2),
        pred_size_3d, tgt_size_3d.reshape(N, 3),
        pred_heading,
    ], axis=1)

    hm_spec = pl.BlockSpec((B, NC, _HB, W), lambda i: (0, 0, i, 0))
    out = pl.pallas_call(
        _tc_body,
        grid=(_GRID,),
        in_specs=[
            hm_spec, hm_spec,
            _full2((32, 128)), _full2((32, 128)), _full2((32, 128)),
            pl.BlockSpec((9, N, 49), lambda i: (0, 0, 0)),
            _full2((N, 39)),
        ],
        out_specs=pl.BlockSpec(memory_space=pltpu.SMEM),
        out_shape=jax.ShapeDtypeStruct((1, 1), jnp.float32),
        scratch_shapes=[pltpu.SMEM((3,), jnp.float32)],
    )(pred_heatmap, tgt_heatmap, g2d, t2d, m2d, a_pack, b_pack)
    return jnp.reshape(out, ())
